# initial kernel scaffold (unmeasured)
import jax
import jax.numpy as jnp
from jax import lax
from jax.experimental import pallas as pl
from jax.experimental.pallas import tpu as pltpu

N_DEV = 4
SQ = 1024
SKV_LOC = 1024
HQ = 8
DH = 128
DM = 1024
SCALE = 0.08838834764831843
NEG = -1e9


def kernel(x, Wq, K_ext, V_ext, Wo):
    def body(x_ref, wq_ref, k_ref, v_ref, wo_ref, out_ref,
             ctx_comm, stats_comm, send_sems, recv_sems):
        my_pos = lax.axis_index("i")

        x2 = x_ref[0]
        q2 = jnp.dot(x2, wq_ref[...], preferred_element_type=jnp.float32)

        qi = lax.broadcasted_iota(jnp.int32, (SQ, SKV_LOC), 0)
        kj = lax.broadcasted_iota(jnp.int32, (SQ, SKV_LOC), 1) + my_pos * SKV_LOC
        mask = (jnp.abs(qi - kj) <= 128) | (kj < 32) | (qi < 32)

        ctx_parts = []
        m_parts = []
        l_parts = []
        for h in range(HQ):
            qh = q2[:, h * DH:(h + 1) * DH]
            kh = k_ref[0, :, h, :]
            vh = v_ref[0, :, h, :]
            s = lax.dot_general(qh, kh, (((1,), (1,)), ((), ())),
                                preferred_element_type=jnp.float32) * SCALE
            s = jnp.where(mask, s, NEG)
            m = jnp.max(s, axis=1, keepdims=True)
            w = jnp.exp(s - m)
            l = jnp.sum(w, axis=1, keepdims=True)
            ctx_parts.append(jnp.dot(w, vh, preferred_element_type=jnp.float32))
            m_parts.append(m)
            l_parts.append(l)

        ctx_comm[my_pos] = jnp.concatenate(ctx_parts, axis=1)
        stats_comm[my_pos] = jnp.concatenate(m_parts + l_parts, axis=1)

        sends = []
        for off in range(1, N_DEV):
            p = lax.rem(my_pos + off, N_DEV)
            for j, buf in ((0, ctx_comm), (1, stats_comm)):
                rdma = pltpu.make_async_remote_copy(
                    src_ref=buf.at[my_pos],
                    dst_ref=buf.at[my_pos],
                    send_sem=send_sems.at[p, j],
                    recv_sem=recv_sems.at[my_pos, j],
                    device_id=(p,),
                    device_id_type=pl.DeviceIdType.MESH,
                )
                rdma.start()
                sends.append(rdma)

        for off in range(1, N_DEV):
            p = lax.rem(my_pos + off, N_DEV)
            for j, buf in ((0, ctx_comm), (1, stats_comm)):
                recv = pltpu.make_async_remote_copy(
                    src_ref=buf.at[p],
                    dst_ref=buf.at[p],
                    send_sem=send_sems.at[p, j],
                    recv_sem=recv_sems.at[p, j],
                    device_id=(p,),
                    device_id_type=pl.DeviceIdType.MESH,
                )
                recv.wait_recv()

        stats = stats_comm[...]
        m_all = stats[:, :, 0:HQ]
        l_all = stats[:, :, HQ:2 * HQ]
        mx = jnp.max(m_all, axis=0)
        f = jnp.exp(m_all - mx[None])
        lsum = jnp.sum(l_all * f, axis=0)

        outs = []
        for h in range(HQ):
            acc = None
            for d in range(N_DEV):
                term = ctx_comm[d, :, h * DH:(h + 1) * DH] * f[d, :, h:h + 1]
                acc = term if acc is None else acc + term
            outs.append(acc / lsum[:, h:h + 1])
        ctxn = jnp.concatenate(outs, axis=1)

        out_ref[0] = jnp.dot(ctxn, wo_ref[...], preferred_element_type=jnp.float32)

        for rdma in sends:
            rdma.wait_send()

    return pl.pallas_call(
        body,
        out_shape=jax.ShapeDtypeStruct((1, SQ, DM), jnp.float32),
        in_specs=[pl.BlockSpec(memory_space=pltpu.VMEM)] * 5,
        out_specs=pl.BlockSpec(memory_space=pltpu.VMEM),
        scratch_shapes=[
            pltpu.VMEM((N_DEV, SQ, DM), jnp.float32),
            pltpu.VMEM((N_DEV, SQ, 2 * HQ), jnp.float32),
            pltpu.SemaphoreType.DMA((N_DEV, 2)),
            pltpu.SemaphoreType.DMA((N_DEV, 2)),
        ],
        compiler_params=pltpu.CompilerParams(collective_id=0),
    )(x, Wq, K_ext, V_ext, Wo)


# baseline (device time: 168645 ns/iter reference)
import jax
import jax.numpy as jnp
from jax import lax
from jax.experimental import pallas as pl
from jax.experimental.pallas import tpu as pltpu

N_DEV = 4
SQ = 1024
SKV_LOC = 1024
HQ = 8
DH = 128
DM = 1024
SCALE = 0.08838834764831843
NEG = -1e9


def _attn_body(q_ref, k_ref, v_ref, ctx_ref, stats_ref):
    my_pos = lax.axis_index("i")

    qi = lax.broadcasted_iota(jnp.int32, (SQ, SKV_LOC), 0)
    kj = lax.broadcasted_iota(jnp.int32, (SQ, SKV_LOC), 1) + my_pos * SKV_LOC
    mask = (jnp.abs(qi - kj) <= 128) | (kj < 32) | (qi < 32)

    m_parts = []
    l_parts = []
    for h in range(HQ):
        qh = q_ref[:, h * DH:(h + 1) * DH]
        kh = k_ref[0, :, h, :]
        vh = v_ref[0, :, h, :]
        s = lax.dot_general(qh, kh, (((1,), (1,)), ((), ())),
                            preferred_element_type=jnp.float32) * SCALE
        s = jnp.where(mask, s, NEG)
        m = jnp.max(s, axis=1, keepdims=True)
        w = jnp.exp(s - m)
        l = jnp.sum(w, axis=1, keepdims=True)
        ctx_ref[:, h * DH:(h + 1) * DH] = jnp.dot(
            w, vh, preferred_element_type=jnp.float32)
        m_parts.append(m)
        l_parts.append(l)

    stats_ref[...] = jnp.concatenate(m_parts + l_parts, axis=1)


def _merge_body(ctx_ref, stats_ref, out_ref,
                ctx_comm, stats_comm, send_sems, recv_sems):
    my_pos = lax.axis_index("i")

    sends = []
    for off in range(1, N_DEV):
        p = lax.rem(my_pos + off, N_DEV)
        for j, (src, buf) in ((0, (ctx_ref, ctx_comm)),
                              (1, (stats_ref, stats_comm))):
            rdma = pltpu.make_async_remote_copy(
                src_ref=src,
                dst_ref=buf.at[off - 1],
                send_sem=send_sems.at[off - 1, j],
                recv_sem=recv_sems.at[off - 1, j],
                device_id=(p,),
                device_id_type=pl.DeviceIdType.MESH,
            )
            rdma.start()
            sends.append(rdma)

    for off in range(1, N_DEV):
        for j, (src, buf) in ((0, (ctx_ref, ctx_comm)),
                              (1, (stats_ref, stats_comm))):
            recv = pltpu.make_async_remote_copy(
                src_ref=src,
                dst_ref=buf.at[off - 1],
                send_sem=send_sems.at[off - 1, j],
                recv_sem=recv_sems.at[off - 1, j],
                device_id=(my_pos,),
                device_id_type=pl.DeviceIdType.MESH,
            )
            recv.wait_recv()

    for h in range(HQ):
        hs = slice(h * DH, (h + 1) * DH)
        ms = [stats_comm[s_, :, h:h + 1] for s_ in range(N_DEV - 1)]
        ms.append(stats_ref[:, h:h + 1])
        ls = [stats_comm[s_, :, HQ + h:HQ + h + 1] for s_ in range(N_DEV - 1)]
        ls.append(stats_ref[:, HQ + h:HQ + h + 1])
        mx = jnp.maximum(jnp.maximum(ms[0], ms[1]),
                         jnp.maximum(ms[2], ms[3]))
        fs = [jnp.exp(m_ - mx) for m_ in ms]
        lsum = sum(l_ * f_ for l_, f_ in zip(ls, fs))
        acc = ctx_ref[:, hs] * fs[N_DEV - 1]
        for s_ in range(N_DEV - 1):
            acc = acc + ctx_comm[s_, :, hs] * fs[s_]
        out_ref[:, hs] = acc / lsum

    for rdma in sends:
        rdma.wait_send()


def kernel(x, Wq, K_ext, V_ext, Wo):
    q2 = jnp.dot(x[0], Wq, preferred_element_type=jnp.float32)

    ctx_own, stats_own = pl.pallas_call(
        _attn_body,
        out_shape=[
            jax.ShapeDtypeStruct((SQ, DM), jnp.float32),
            jax.ShapeDtypeStruct((SQ, 2 * HQ), jnp.float32),
        ],
        in_specs=[pl.BlockSpec(memory_space=pltpu.VMEM)] * 3,
        out_specs=[pl.BlockSpec(memory_space=pltpu.VMEM)] * 2,
        compiler_params=pltpu.CompilerParams(
            vmem_limit_bytes=60 * 1024 * 1024,
        ),
    )(q2, K_ext, V_ext)

    ctxn = pl.pallas_call(
        _merge_body,
        out_shape=jax.ShapeDtypeStruct((SQ, DM), jnp.float32),
        in_specs=[pl.BlockSpec(memory_space=pltpu.VMEM)] * 2,
        out_specs=pl.BlockSpec(memory_space=pltpu.VMEM),
        scratch_shapes=[
            pltpu.VMEM((N_DEV - 1, SQ, DM), jnp.float32),
            pltpu.VMEM((N_DEV - 1, SQ, 2 * HQ), jnp.float32),
            pltpu.SemaphoreType.DMA((N_DEV - 1, 2)),
            pltpu.SemaphoreType.DMA((N_DEV - 1, 2)),
        ],
        compiler_params=pltpu.CompilerParams(
            vmem_limit_bytes=60 * 1024 * 1024,
        ),
    )(ctx_own, stats_own)

    return jnp.dot(ctxn, Wo, preferred_element_type=jnp.float32)[None]


# device time: 88458 ns/iter; 1.9065x vs baseline; 1.9065x over previous
import jax
import jax.numpy as jnp
from jax import lax
from jax.experimental import pallas as pl
from jax.experimental.pallas import tpu as pltpu

N_DEV = 4
SQ = 1024
SKV_LOC = 1024
HQ = 8
DH = 128
DM = 1024
SCALE = 0.08838834764831843
NEG = -1e9

HALF = SQ // 2
R0 = 32
R2 = 896


def _attn_body(q_ref, k_ref, v_ref, ctx_ref, stats_ref):
    my_pos = lax.axis_index("i")

    qi = lax.broadcasted_iota(jnp.int32, (SQ, SKV_LOC), 0)
    kj = lax.broadcasted_iota(jnp.int32, (SQ, SKV_LOC), 1) + my_pos * SKV_LOC
    mask = (jnp.abs(qi - kj) <= 128) | (kj < 32) | (qi < 32)

    m_parts = []
    l_parts = []
    for h in range(HQ):
        qh = q_ref[:, h * DH:(h + 1) * DH]
        kh = k_ref[0, :, h, :]
        vh = v_ref[0, :, h, :]
        s = lax.dot_general(qh, kh, (((1,), (1,)), ((), ())),
                            preferred_element_type=jnp.float32) * SCALE
        s = jnp.where(mask, s, NEG)
        m = jnp.max(s, axis=1, keepdims=True)
        w = jnp.exp(s - m)
        l = jnp.sum(w, axis=1, keepdims=True)
        ctx_ref[:, h * DH:(h + 1) * DH] = jnp.dot(
            w, vh, preferred_element_type=jnp.float32).astype(jnp.bfloat16)
        m_parts.append(m)
        l_parts.append(l)

    stats_ref[...] = jnp.concatenate(m_parts + l_parts, axis=1)


def _merge_body(ctx_ref, stats_ref, out_ref,
                big, sm32, sm128, stats_all,
                big_recv, sm32_recv, sm128_recv, stats_recv, send_sems):
    my_pos = lax.axis_index("i")

    def rdma(src, dst, ssem, rsem, dev):
        return pltpu.make_async_remote_copy(
            src_ref=src, dst_ref=dst, send_sem=ssem, recv_sem=rsem,
            device_id=(dev,), device_id_type=pl.DeviceIdType.MESH)

    stats_all[my_pos] = stats_ref[...]

    a_lo = ctx_ref.at[pl.ds(0, HALF)]
    a_hi = ctx_ref.at[pl.ds(HALF, HALF)]
    b_lo = big.at[pl.ds(0, HALF)]
    b_hi = big.at[pl.ds(HALF, HALF)]

    @pl.when(my_pos == 0)
    def _():
        sends = [
            rdma(a_lo, b_lo, send_sems.at[0], big_recv.at[0], 1),
            rdma(a_hi, b_hi, send_sems.at[1], big_recv.at[1], 3),
            rdma(a_hi, b_hi, send_sems.at[2], big_recv.at[1], 1),
            rdma(a_lo, b_lo, send_sems.at[3], big_recv.at[0], 3),
            rdma(stats_ref, stats_all.at[0], send_sems.at[4], stats_recv.at[0], 1),
            rdma(stats_ref, stats_all.at[0], send_sems.at[5], stats_recv.at[0], 2),
            rdma(stats_ref, stats_all.at[0], send_sems.at[6], stats_recv.at[0], 3),
        ]
        for r in sends:
            r.start()
        big[...] = ctx_ref[...]
        rdma(sm32.at[0], sm32.at[0], send_sems.at[7], sm32_recv.at[0], 0).wait_recv()
        rdma(sm32.at[1], sm32.at[1], send_sems.at[7], sm32_recv.at[1], 0).wait_recv()
        rdma(sm32.at[2], sm32.at[2], send_sems.at[7], sm32_recv.at[2], 0).wait_recv()
        rdma(sm128, sm128, send_sems.at[7], sm128_recv.at[0], 0).wait_recv()
        for o in (1, 2, 3):
            rdma(stats_ref, stats_all.at[o], send_sems.at[7],
                 stats_recv.at[o], 0).wait_recv()
        for r in sends:
            r.wait_send()

    @pl.when(my_pos == 1)
    def _():
        own32 = ctx_ref.at[pl.ds(0, R0)]
        own128 = ctx_ref.at[pl.ds(R2, SQ - R2)]
        sends = [
            rdma(own32, sm32.at[0], send_sems.at[0], sm32_recv.at[0], 0),
            rdma(own32, sm32.at[0], send_sems.at[1], sm32_recv.at[0], 2),
            rdma(own32, sm32.at[0], send_sems.at[2], sm32_recv.at[0], 3),
            rdma(own128, sm128, send_sems.at[3], sm128_recv.at[0], 0),
            rdma(own128, sm128, send_sems.at[4], sm128_recv.at[0], 2),
            rdma(own128, sm128, send_sems.at[5], sm128_recv.at[0], 3),
            rdma(stats_ref, stats_all.at[1], send_sems.at[6], stats_recv.at[1], 0),
            rdma(stats_ref, stats_all.at[1], send_sems.at[7], stats_recv.at[1], 2),
            rdma(stats_ref, stats_all.at[1], send_sems.at[8], stats_recv.at[1], 3),
        ]
        for r in sends:
            r.start()
        sm32[0] = ctx_ref[0:R0, :]
        sm128[...] = ctx_ref[R2:SQ, :]
        rdma(b_lo, b_lo, send_sems.at[9], big_recv.at[0], 0).wait_recv()
        fwd = rdma(b_lo, b_lo, send_sems.at[9], big_recv.at[0], 2)
        fwd.start()
        rdma(b_hi, b_hi, send_sems.at[10], big_recv.at[1], 0).wait_recv()
        rdma(sm32.at[1], sm32.at[1], send_sems.at[11], sm32_recv.at[1], 0).wait_recv()
        rdma(sm32.at[2], sm32.at[2], send_sems.at[11], sm32_recv.at[2], 0).wait_recv()
        for o in (0, 2, 3):
            rdma(stats_ref, stats_all.at[o], send_sems.at[11],
                 stats_recv.at[o], 0).wait_recv()
        fwd.wait_send()
        for r in sends:
            r.wait_send()

    @pl.when(my_pos == 2)
    def _():
        own32 = ctx_ref.at[pl.ds(0, R0)]
        sends = [
            rdma(own32, sm32.at[1], send_sems.at[0], sm32_recv.at[1], 0),
            rdma(own32, sm32.at[1], send_sems.at[1], sm32_recv.at[1], 1),
            rdma(own32, sm32.at[1], send_sems.at[2], sm32_recv.at[1], 3),
            rdma(stats_ref, stats_all.at[2], send_sems.at[3], stats_recv.at[2], 0),
            rdma(stats_ref, stats_all.at[2], send_sems.at[4], stats_recv.at[2], 1),
            rdma(stats_ref, stats_all.at[2], send_sems.at[5], stats_recv.at[2], 3),
        ]
        for r in sends:
            r.start()
        sm32[1] = ctx_ref[0:R0, :]
        rdma(b_lo, b_lo, send_sems.at[6], big_recv.at[0], 0).wait_recv()
        rdma(b_hi, b_hi, send_sems.at[6], big_recv.at[1], 0).wait_recv()
        rdma(sm32.at[0], sm32.at[0], send_sems.at[6], sm32_recv.at[0], 0).wait_recv()
        rdma(sm32.at[2], sm32.at[2], send_sems.at[6], sm32_recv.at[2], 0).wait_recv()
        rdma(sm128, sm128, send_sems.at[6], sm128_recv.at[0], 0).wait_recv()
        for o in (0, 1, 3):
            rdma(stats_ref, stats_all.at[o], send_sems.at[6],
                 stats_recv.at[o], 0).wait_recv()
        for r in sends:
            r.wait_send()

    @pl.when(my_pos == 3)
    def _():
        own32 = ctx_ref.at[pl.ds(0, R0)]
        sends = [
            rdma(own32, sm32.at[2], send_sems.at[0], sm32_recv.at[2], 0),
            rdma(own32, sm32.at[2], send_sems.at[1], sm32_recv.at[2], 1),
            rdma(own32, sm32.at[2], send_sems.at[2], sm32_recv.at[2], 2),
            rdma(stats_ref, stats_all.at[3], send_sems.at[3], stats_recv.at[3], 0),
            rdma(stats_ref, stats_all.at[3], send_sems.at[4], stats_recv.at[3], 1),
            rdma(stats_ref, stats_all.at[3], send_sems.at[5], stats_recv.at[3], 2),
        ]
        for r in sends:
            r.start()
        sm32[2] = ctx_ref[0:R0, :]
        rdma(b_hi, b_hi, send_sems.at[9], big_recv.at[1], 0).wait_recv()
        fwd = rdma(b_hi, b_hi, send_sems.at[9], big_recv.at[1], 2)
        fwd.start()
        rdma(b_lo, b_lo, send_sems.at[10], big_recv.at[0], 0).wait_recv()
        rdma(sm32.at[0], sm32.at[0], send_sems.at[11], sm32_recv.at[0], 0).wait_recv()
        rdma(sm32.at[1], sm32.at[1], send_sems.at[11], sm32_recv.at[1], 0).wait_recv()
        rdma(sm128, sm128, send_sems.at[11], sm128_recv.at[0], 0).wait_recv()
        for o in (0, 1, 2):
            rdma(stats_ref, stats_all.at[o], send_sems.at[11],
                 stats_recv.at[o], 0).wait_recv()
        fwd.wait_send()
        for r in sends:
            r.wait_send()

    f32 = jnp.float32
    for h in range(HQ):
        hs = slice(h * DH, (h + 1) * DH)
        mc = h
        lc = HQ + h

        l0 = stats_all[0, R0:R2, lc:lc + 1]
        out_ref[R0:R2, hs] = big[R0:R2, hs].astype(f32) / l0

        ms = [stats_all[d, 0:R0, mc:mc + 1] for d in range(N_DEV)]
        ls = [stats_all[d, 0:R0, lc:lc + 1] for d in range(N_DEV)]
        mx = jnp.maximum(jnp.maximum(ms[0], ms[1]), jnp.maximum(ms[2], ms[3]))
        fs = [jnp.exp(m_ - mx) for m_ in ms]
        lsum = sum(l_ * f_ for l_, f_ in zip(ls, fs))
        acc = big[0:R0, hs].astype(f32) * fs[0]
        for k in range(3):
            acc = acc + sm32[k, :, hs].astype(f32) * fs[k + 1]
        out_ref[0:R0, hs] = acc / lsum

        m0 = stats_all[0, R2:SQ, mc:mc + 1]
        m1 = stats_all[1, R2:SQ, mc:mc + 1]
        l0 = stats_all[0, R2:SQ, lc:lc + 1]
        l1 = stats_all[1, R2:SQ, lc:lc + 1]
        mx2 = jnp.maximum(m0, m1)
        f0 = jnp.exp(m0 - mx2)
        f1 = jnp.exp(m1 - mx2)
        acc2 = (big[R2:SQ, hs].astype(f32) * f0
                + sm128[:, hs].astype(f32) * f1)
        out_ref[R2:SQ, hs] = acc2 / (l0 * f0 + l1 * f1)


def kernel(x, Wq, K_ext, V_ext, Wo):
    q2 = jnp.dot(x[0], Wq, preferred_element_type=jnp.float32)

    ctx_own, stats_own = pl.pallas_call(
        _attn_body,
        out_shape=[
            jax.ShapeDtypeStruct((SQ, DM), jnp.bfloat16),
            jax.ShapeDtypeStruct((SQ, 2 * HQ), jnp.float32),
        ],
        in_specs=[pl.BlockSpec(memory_space=pltpu.VMEM)] * 3,
        out_specs=[pl.BlockSpec(memory_space=pltpu.VMEM)] * 2,
        compiler_params=pltpu.CompilerParams(
            vmem_limit_bytes=60 * 1024 * 1024,
        ),
    )(q2, K_ext, V_ext)

    ctxn = pl.pallas_call(
        _merge_body,
        out_shape=jax.ShapeDtypeStruct((SQ, DM), jnp.float32),
        in_specs=[pl.BlockSpec(memory_space=pltpu.VMEM)] * 2,
        out_specs=pl.BlockSpec(memory_space=pltpu.VMEM),
        scratch_shapes=[
            pltpu.VMEM((SQ, DM), jnp.bfloat16),
            pltpu.VMEM((3, R0, DM), jnp.bfloat16),
            pltpu.VMEM((SQ - R2, DM), jnp.bfloat16),
            pltpu.VMEM((N_DEV, SQ, 2 * HQ), jnp.float32),
            pltpu.SemaphoreType.DMA((2,)),
            pltpu.SemaphoreType.DMA((3,)),
            pltpu.SemaphoreType.DMA((1,)),
            pltpu.SemaphoreType.DMA((N_DEV,)),
            pltpu.SemaphoreType.DMA((12,)),
        ],
        compiler_params=pltpu.CompilerParams(
            vmem_limit_bytes=60 * 1024 * 1024,
        ),
    )(ctx_own, stats_own)

    return jnp.dot(ctxn, Wo, preferred_element_type=jnp.float32)[None]
